# Initial kernel scaffold; baseline (speedup 1.0000x reference)
import jax
import jax.numpy as jnp
from jax.experimental import pallas as pl


def kernel(x, edge_index, batch, l0_w1, l0_b1, l0_w2, l0_b2, l1_w1, l1_b1, l1_w2, l1_b2, l2_w1, l2_b1, l2_w2, l2_b2):
    raise NotImplementedError("write your pallas kernel here")


# speedup vs baseline: 5.8998x; 5.8998x over previous
import functools

import jax
import jax.numpy as jnp
from jax import lax
from jax.experimental import pallas as pl
from jax.experimental.pallas import tpu as pltpu
from jax.experimental.pallas import tpu_sc as plsc

N_NODES = 10000
N_EDGES = 320000
N_GRAPHS = 64
HIDDEN = 256
HALF = 128

NS = 16
CHUNK = 125
RPT = 624
RTAIL = N_NODES - NS * RPT

NCHUNK0 = N_EDGES // 2 // NS // CHUNK
NCHUNK12 = N_EDGES // NS // CHUNK
IBLK = 16

BLOCK_ROWS = 1000
GRID = N_NODES // BLOCK_ROWS


def _make_agg(nchunk):
    mesh = plsc.VectorSubcoreMesh(core_axis_name="c", subcore_axis_name="s")

    @functools.partial(
        pl.kernel,
        mesh=mesh,
        out_type=jax.ShapeDtypeStruct((2, N_NODES, HALF), jnp.float32),
        scratch_types=[
            pltpu.VMEM((IBLK, CHUNK), jnp.int32),
            pltpu.VMEM((IBLK, CHUNK), jnp.int32),
            pltpu.VMEM((CHUNK, HALF), jnp.float32),
            pltpu.VMEM_SHARED((N_NODES, HALF), jnp.float32),
            pltpu.SemaphoreType.DMA,
        ],
    )
    def agg_kernel(table_hbm, init_hbm, src_hbm, dst_hbm, out_hbm,
                   src_v, dst_v, buf, accum, sem):
        c = lax.axis_index("c")
        s = lax.axis_index("s")
        pltpu.sync_copy(init_hbm.at[c, pl.ds(s * RPT, RPT)],
                        accum.at[pl.ds(s * RPT, RPT)])

        @pl.when(s == NS - 1)
        def _():
            pltpu.sync_copy(init_hbm.at[c, pl.ds(NS * RPT, RTAIL)],
                            accum.at[pl.ds(NS * RPT, RTAIL)])

        plsc.subcore_barrier()

        def blk(k, carry):
            pltpu.sync_copy(src_hbm.at[c, s, pl.ds(k * IBLK, IBLK)], src_v)
            pltpu.sync_copy(dst_hbm.at[c, s, pl.ds(k * IBLK, IBLK)], dst_v)

            def body(j, carry2):
                pltpu.async_copy(table_hbm.at[c].at[src_v.at[j]], buf,
                                 sem).wait()
                pltpu.sync_copy(buf, accum.at[dst_v.at[j]], add=True)
                return carry2

            return lax.fori_loop(0, IBLK, body, carry)

        lax.fori_loop(0, nchunk // IBLK, blk, 0)
        plsc.subcore_barrier()
        pltpu.sync_copy(accum.at[pl.ds(s * RPT, RPT)],
                        out_hbm.at[c, pl.ds(s * RPT, RPT)])

        @pl.when(s == NS - 1)
        def _():
            pltpu.sync_copy(accum.at[pl.ds(NS * RPT, RTAIL)],
                            out_hbm.at[c, pl.ds(NS * RPT, RTAIL)])

    return agg_kernel


def _mlp_core(z_ref, w1_ref, mode):
    if mode == "sum":
        z = z_ref[0] + z_ref[1]
        return jnp.dot(z, w1_ref[...], preferred_element_type=jnp.float32)
    a = jnp.dot(z_ref[0], w1_ref[0:HALF, :],
                preferred_element_type=jnp.float32)
    a += jnp.dot(z_ref[1], w1_ref[HALF:2 * HALF, :],
                 preferred_element_type=jnp.float32)
    return a


def _make_mlp(d_in, mode):
    def body(z_ref, w1_ref, b1_ref, w2_ref, b2_ref, out_ref):
        a = jnp.maximum(_mlp_core(z_ref, w1_ref, mode) + b1_ref[...], 0.0)
        h = jnp.dot(a, w2_ref[...], preferred_element_type=jnp.float32)
        h = jnp.maximum(h + b2_ref[...], 0.0)
        out_ref[0] = h[:, 0:HALF]
        out_ref[1] = h[:, HALF:2 * HALF]

    return pl.pallas_call(
        body,
        grid=(GRID,),
        in_specs=[
            pl.BlockSpec((2, BLOCK_ROWS, HALF), lambda i: (0, i, 0)),
            pl.BlockSpec((d_in, HIDDEN), lambda i: (0, 0)),
            pl.BlockSpec((1, HIDDEN), lambda i: (0, 0)),
            pl.BlockSpec((HIDDEN, HIDDEN), lambda i: (0, 0)),
            pl.BlockSpec((1, HIDDEN), lambda i: (0, 0)),
        ],
        out_specs=pl.BlockSpec((2, BLOCK_ROWS, HALF), lambda i: (0, i, 0)),
        out_shape=jax.ShapeDtypeStruct((2, N_NODES, HALF), jnp.float32),
    )


def _mlp_pool_body(z_ref, w1_ref, b1_ref, w2_ref, b2_ref, batch_ref, g_ref,
                   sums_ref, cnts_ref):
    i = pl.program_id(0)

    @pl.when(i == 0)
    def _():
        sums_ref[...] = jnp.zeros_like(sums_ref)
        cnts_ref[...] = jnp.zeros_like(cnts_ref)

    a = jnp.maximum(_mlp_core(z_ref, w1_ref, "split") + b1_ref[...], 0.0)
    h = jnp.dot(a, w2_ref[...], preferred_element_type=jnp.float32)
    h = jnp.maximum(h + b2_ref[...], 0.0)

    b = batch_ref[0, 0, :]
    gids = lax.broadcasted_iota(jnp.int32, (N_GRAPHS, BLOCK_ROWS), 0)
    onehot = (b[None, :] == gids).astype(jnp.float32)
    sums_ref[...] += jnp.dot(onehot, h, preferred_element_type=jnp.float32)
    cnts_ref[...] += jnp.sum(onehot, axis=1, keepdims=True)

    @pl.when(i == GRID - 1)
    def _():
        g_ref[...] = sums_ref[...] / jnp.maximum(cnts_ref[:, :1], 1.0)


def _make_mlp_pool():
    return pl.pallas_call(
        _mlp_pool_body,
        grid=(GRID,),
        in_specs=[
            pl.BlockSpec((2, BLOCK_ROWS, HALF), lambda i: (0, i, 0)),
            pl.BlockSpec((HIDDEN, HIDDEN), lambda i: (0, 0)),
            pl.BlockSpec((1, HIDDEN), lambda i: (0, 0)),
            pl.BlockSpec((HIDDEN, HIDDEN), lambda i: (0, 0)),
            pl.BlockSpec((1, HIDDEN), lambda i: (0, 0)),
            pl.BlockSpec((1, 1, BLOCK_ROWS), lambda i: (i, 0, 0)),
        ],
        out_specs=pl.BlockSpec((N_GRAPHS, HIDDEN), lambda i: (0, 0)),
        out_shape=jax.ShapeDtypeStruct((N_GRAPHS, HIDDEN), jnp.float32),
        scratch_shapes=[
            pltpu.VMEM((N_GRAPHS, HIDDEN), jnp.float32),
            pltpu.VMEM((N_GRAPHS, 1), jnp.float32),
        ],
    )


_AGG0 = _make_agg(NCHUNK0)
_AGG12 = _make_agg(NCHUNK12)
_MLP0 = _make_mlp(128, "sum")
_MLP1 = _make_mlp(256, "split")
_MLP_POOL = _make_mlp_pool()


def kernel(x, edge_index, batch,
           l0_w1, l0_b1, l0_w2, l0_b2,
           l1_w1, l1_b1, l1_w2, l1_b2,
           l2_w1, l2_b1, l2_w2, l2_b2):
    ei = edge_index.astype(jnp.int32)
    src0 = ei[0].reshape(2, NS, NCHUNK0, CHUNK)
    dst0 = ei[1].reshape(2, NS, NCHUNK0, CHUNK)
    src12 = jnp.broadcast_to(ei[0].reshape(1, NS, NCHUNK12, CHUNK),
                             (2, NS, NCHUNK12, CHUNK))
    dst12 = jnp.broadcast_to(ei[1].reshape(1, NS, NCHUNK12, CHUNK),
                             (2, NS, NCHUNK12, CHUNK))
    batch_r = batch.astype(jnp.int32).reshape(GRID, 1, BLOCK_ROWS)

    x2 = jnp.broadcast_to(x[None], (2, N_NODES, HALF))
    xz = jnp.concatenate([x[None], jnp.zeros((1, N_NODES, HALF), x.dtype)], 0)

    p = _AGG0(x2, xz, src0, dst0)
    h = _MLP0(p, l0_w1, l0_b1.reshape(1, HIDDEN),
              l0_w2, l0_b2.reshape(1, HIDDEN))
    z = _AGG12(h, h, src12, dst12)
    h = _MLP1(z, l1_w1, l1_b1.reshape(1, HIDDEN),
              l1_w2, l1_b2.reshape(1, HIDDEN))
    z = _AGG12(h, h, src12, dst12)
    g = _MLP_POOL(z, l2_w1, l2_b1.reshape(1, HIDDEN),
                  l2_w2, l2_b2.reshape(1, HIDDEN), batch_r)
    return g


# speedup vs baseline: 9.2882x; 1.5743x over previous
import functools

import jax
import jax.numpy as jnp
from jax import lax
from jax.experimental import pallas as pl
from jax.experimental.pallas import tpu as pltpu
from jax.experimental.pallas import tpu_sc as plsc

N_NODES = 10000
N_EDGES = 320000
N_GRAPHS = 64
HIDDEN = 256
HALF = 128

NS = 16
CHUNK = 125
RPT = 624
RTAIL = N_NODES - NS * RPT

NCHUNK0 = N_EDGES // 2 // NS // CHUNK
NCHUNK12 = N_EDGES // NS // CHUNK
IBLK = 16

BLOCK_ROWS = 1000
GRID = N_NODES // BLOCK_ROWS


def _make_agg(nchunk):
    mesh = plsc.VectorSubcoreMesh(core_axis_name="c", subcore_axis_name="s")

    nblk = nchunk // IBLK

    @functools.partial(
        pl.kernel,
        mesh=mesh,
        out_type=jax.ShapeDtypeStruct((2, N_NODES, HALF), jnp.float32),
        scratch_types=[
            pltpu.VMEM((2 * IBLK, CHUNK), jnp.int32),
            pltpu.VMEM((2 * IBLK, CHUNK), jnp.int32),
            pltpu.VMEM((2, CHUNK, HALF), jnp.float32),
            pltpu.VMEM_SHARED((N_NODES, HALF), jnp.float32),
            pltpu.SemaphoreType.DMA,
            pltpu.SemaphoreType.DMA,
        ],
    )
    def agg_kernel(table_hbm, init_hbm, src_hbm, dst_hbm, out_hbm,
                   src_v, dst_v, buf, accum, sem0, sem1):
        c = lax.axis_index("c")
        s = lax.axis_index("s")
        gsem = (sem0, sem1)
        pltpu.sync_copy(init_hbm.at[c, pl.ds(s * RPT, RPT)],
                        accum.at[pl.ds(s * RPT, RPT)])

        @pl.when(s == NS - 1)
        def _():
            pltpu.sync_copy(init_hbm.at[c, pl.ds(NS * RPT, RTAIL)],
                            accum.at[pl.ds(NS * RPT, RTAIL)])

        plsc.subcore_barrier()

        def fire_gather(idx_row, p):
            pltpu.async_copy(table_hbm.at[c].at[idx_row], buf.at[p], gsem[p])

        def wait_gather(p):
            pltpu.make_async_copy(table_hbm.at[c].at[src_v.at[0]],
                                  buf.at[p], gsem[p]).wait()

        pltpu.sync_copy(src_hbm.at[c, s, pl.ds(0, IBLK)],
                        src_v.at[pl.ds(0, IBLK)])
        pltpu.sync_copy(dst_hbm.at[c, s, pl.ds(0, IBLK)],
                        dst_v.at[pl.ds(0, IBLK)])
        fire_gather(src_v.at[0], 0)

        def blk(k, carry):
            co = pl.multiple_of((k % 2) * IBLK, IBLK)
            no = pl.multiple_of(IBLK - co, IBLK)

            @pl.when(k < nblk - 1)
            def _():
                pltpu.sync_copy(
                    src_hbm.at[c, s, pl.ds((k + 1) * IBLK, IBLK)],
                    src_v.at[pl.ds(no, IBLK)])
                pltpu.sync_copy(
                    dst_hbm.at[c, s, pl.ds((k + 1) * IBLK, IBLK)],
                    dst_v.at[pl.ds(no, IBLK)])

            for j2 in range(IBLK):
                p = j2 & 1
                pn = p ^ 1
                if j2 < IBLK - 1:
                    fire_gather(src_v.at[co + j2 + 1], pn)
                else:
                    @pl.when(k < nblk - 1)
                    def _():
                        fire_gather(src_v.at[no], pn)
                wait_gather(p)
                pltpu.sync_copy(buf.at[p], accum.at[dst_v.at[co + j2]],
                                add=True)
            return carry

        lax.fori_loop(0, nblk, blk, 0)
        plsc.subcore_barrier()
        pltpu.sync_copy(accum.at[pl.ds(s * RPT, RPT)],
                        out_hbm.at[c, pl.ds(s * RPT, RPT)])

        @pl.when(s == NS - 1)
        def _():
            pltpu.sync_copy(accum.at[pl.ds(NS * RPT, RTAIL)],
                            out_hbm.at[c, pl.ds(NS * RPT, RTAIL)])

    return agg_kernel


def _mlp_core(z_ref, w1_ref, mode):
    if mode == "sum":
        z = z_ref[0] + z_ref[1]
        return jnp.dot(z, w1_ref[...], preferred_element_type=jnp.float32)
    a = jnp.dot(z_ref[0], w1_ref[0:HALF, :],
                preferred_element_type=jnp.float32)
    a += jnp.dot(z_ref[1], w1_ref[HALF:2 * HALF, :],
                 preferred_element_type=jnp.float32)
    return a


def _make_mlp(d_in, mode):
    def body(z_ref, w1_ref, b1_ref, w2_ref, b2_ref, out_ref):
        a = jnp.maximum(_mlp_core(z_ref, w1_ref, mode) + b1_ref[...], 0.0)
        h = jnp.dot(a, w2_ref[...], preferred_element_type=jnp.float32)
        h = jnp.maximum(h + b2_ref[...], 0.0)
        out_ref[0] = h[:, 0:HALF]
        out_ref[1] = h[:, HALF:2 * HALF]

    return pl.pallas_call(
        body,
        grid=(GRID,),
        in_specs=[
            pl.BlockSpec((2, BLOCK_ROWS, HALF), lambda i: (0, i, 0)),
            pl.BlockSpec((d_in, HIDDEN), lambda i: (0, 0)),
            pl.BlockSpec((1, HIDDEN), lambda i: (0, 0)),
            pl.BlockSpec((HIDDEN, HIDDEN), lambda i: (0, 0)),
            pl.BlockSpec((1, HIDDEN), lambda i: (0, 0)),
        ],
        out_specs=pl.BlockSpec((2, BLOCK_ROWS, HALF), lambda i: (0, i, 0)),
        out_shape=jax.ShapeDtypeStruct((2, N_NODES, HALF), jnp.float32),
    )


def _mlp_pool_body(z_ref, w1_ref, b1_ref, w2_ref, b2_ref, batch_ref, g_ref,
                   sums_ref, cnts_ref):
    i = pl.program_id(0)

    @pl.when(i == 0)
    def _():
        sums_ref[...] = jnp.zeros_like(sums_ref)
        cnts_ref[...] = jnp.zeros_like(cnts_ref)

    a = jnp.maximum(_mlp_core(z_ref, w1_ref, "split") + b1_ref[...], 0.0)
    h = jnp.dot(a, w2_ref[...], preferred_element_type=jnp.float32)
    h = jnp.maximum(h + b2_ref[...], 0.0)

    b = batch_ref[0, 0, :]
    gids = lax.broadcasted_iota(jnp.int32, (N_GRAPHS, BLOCK_ROWS), 0)
    onehot = (b[None, :] == gids).astype(jnp.float32)
    sums_ref[...] += jnp.dot(onehot, h, preferred_element_type=jnp.float32)
    cnts_ref[...] += jnp.sum(onehot, axis=1, keepdims=True)

    @pl.when(i == GRID - 1)
    def _():
        g_ref[...] = sums_ref[...] / jnp.maximum(cnts_ref[:, :1], 1.0)


def _make_mlp_pool():
    return pl.pallas_call(
        _mlp_pool_body,
        grid=(GRID,),
        in_specs=[
            pl.BlockSpec((2, BLOCK_ROWS, HALF), lambda i: (0, i, 0)),
            pl.BlockSpec((HIDDEN, HIDDEN), lambda i: (0, 0)),
            pl.BlockSpec((1, HIDDEN), lambda i: (0, 0)),
            pl.BlockSpec((HIDDEN, HIDDEN), lambda i: (0, 0)),
            pl.BlockSpec((1, HIDDEN), lambda i: (0, 0)),
            pl.BlockSpec((1, 1, BLOCK_ROWS), lambda i: (i, 0, 0)),
        ],
        out_specs=pl.BlockSpec((N_GRAPHS, HIDDEN), lambda i: (0, 0)),
        out_shape=jax.ShapeDtypeStruct((N_GRAPHS, HIDDEN), jnp.float32),
        scratch_shapes=[
            pltpu.VMEM((N_GRAPHS, HIDDEN), jnp.float32),
            pltpu.VMEM((N_GRAPHS, 1), jnp.float32),
        ],
    )


_AGG0 = _make_agg(NCHUNK0)
_AGG12 = _make_agg(NCHUNK12)
_MLP0 = _make_mlp(128, "sum")
_MLP1 = _make_mlp(256, "split")
_MLP_POOL = _make_mlp_pool()


def kernel(x, edge_index, batch,
           l0_w1, l0_b1, l0_w2, l0_b2,
           l1_w1, l1_b1, l1_w2, l1_b2,
           l2_w1, l2_b1, l2_w2, l2_b2):
    ei = edge_index.astype(jnp.int32)
    src0 = ei[0].reshape(2, NS, NCHUNK0, CHUNK)
    dst0 = ei[1].reshape(2, NS, NCHUNK0, CHUNK)
    src12 = jnp.broadcast_to(ei[0].reshape(1, NS, NCHUNK12, CHUNK),
                             (2, NS, NCHUNK12, CHUNK))
    dst12 = jnp.broadcast_to(ei[1].reshape(1, NS, NCHUNK12, CHUNK),
                             (2, NS, NCHUNK12, CHUNK))
    batch_r = batch.astype(jnp.int32).reshape(GRID, 1, BLOCK_ROWS)

    x2 = jnp.broadcast_to(x[None], (2, N_NODES, HALF))
    xz = jnp.concatenate([x[None], jnp.zeros((1, N_NODES, HALF), x.dtype)], 0)

    p = _AGG0(x2, xz, src0, dst0)
    h = _MLP0(p, l0_w1, l0_b1.reshape(1, HIDDEN),
              l0_w2, l0_b2.reshape(1, HIDDEN))
    z = _AGG12(h, h, src12, dst12)
    h = _MLP1(z, l1_w1, l1_b1.reshape(1, HIDDEN),
              l1_w2, l1_b2.reshape(1, HIDDEN))
    z = _AGG12(h, h, src12, dst12)
    g = _MLP_POOL(z, l2_w1, l2_b1.reshape(1, HIDDEN),
                  l2_w2, l2_b2.reshape(1, HIDDEN), batch_r)
    return g


# speedup vs baseline: 9.4931x; 1.0221x over previous
import functools

import jax
import jax.numpy as jnp
from jax import lax
from jax.experimental import pallas as pl
from jax.experimental.pallas import tpu as pltpu
from jax.experimental.pallas import tpu_sc as plsc

N_NODES = 10000
N_EDGES = 320000
N_GRAPHS = 64
HIDDEN = 256
HALF = 128

NS = 16
CHUNK = 125
RPT = 624
RTAIL = N_NODES - NS * RPT

NCHUNK0 = N_EDGES // 2 // NS // CHUNK
NCHUNK12 = N_EDGES // NS // CHUNK
IBLK = 16

BLOCK_ROWS = 1000
GRID = N_NODES // BLOCK_ROWS


def _make_agg(nchunk, split_edges):
    mesh = plsc.VectorSubcoreMesh(core_axis_name="c", subcore_axis_name="s")

    nblk = nchunk // IBLK

    @functools.partial(
        pl.kernel,
        mesh=mesh,
        out_type=jax.ShapeDtypeStruct((2, N_NODES, HALF), jnp.float32),
        scratch_types=[
            pltpu.VMEM((2 * IBLK, CHUNK), jnp.int32),
            pltpu.VMEM((2 * IBLK, CHUNK), jnp.int32),
            pltpu.VMEM((2, CHUNK, HALF), jnp.float32),
            pltpu.VMEM_SHARED((N_NODES, HALF), jnp.float32),
            pltpu.SemaphoreType.DMA,
            pltpu.SemaphoreType.DMA,
            pltpu.SemaphoreType.DMA,
            pltpu.SemaphoreType.DMA,
        ],
    )
    def agg_kernel(table_hbm, init_hbm, src_hbm, dst_hbm, out_hbm,
                   src_v, dst_v, buf, accum, gsem0, gsem1, ssem0, ssem1):
        c = lax.axis_index("c")
        s = lax.axis_index("s")
        gsem = (gsem0, gsem1)
        ssem = (ssem0, ssem1)
        tbl = table_hbm if split_edges else table_hbm.at[c]
        ini = init_hbm if split_edges else init_hbm.at[c]

        def eslice(ref, lo, n):
            if split_edges:
                return ref.at[c, s, pl.ds(lo, n)]
            return ref.at[s, pl.ds(lo, n)]

        pltpu.sync_copy(ini.at[pl.ds(s * RPT, RPT)],
                        accum.at[pl.ds(s * RPT, RPT)])

        @pl.when(s == NS - 1)
        def _():
            pltpu.sync_copy(ini.at[pl.ds(NS * RPT, RTAIL)],
                            accum.at[pl.ds(NS * RPT, RTAIL)])

        plsc.subcore_barrier()

        def fire_gather(idx_row, p):
            pltpu.async_copy(tbl.at[idx_row], buf.at[p], gsem[p])

        def wait_gather(p):
            pltpu.make_async_copy(tbl.at[src_v.at[0]], buf.at[p],
                                  gsem[p]).wait()

        def fire_scatter(idx_row, p):
            pltpu.async_copy(buf.at[p], accum.at[idx_row], ssem[p],
                             add=True)

        def wait_scatter(p):
            pltpu.make_async_copy(buf.at[p], accum.at[dst_v.at[0]],
                                  ssem[p]).wait()

        pltpu.sync_copy(eslice(src_hbm, 0, IBLK), src_v.at[pl.ds(0, IBLK)])
        pltpu.sync_copy(eslice(dst_hbm, 0, IBLK), dst_v.at[pl.ds(0, IBLK)])
        fire_gather(src_v.at[0], 0)

        def blk(k, carry):
            co = pl.multiple_of((k % 2) * IBLK, IBLK)
            no = pl.multiple_of(IBLK - co, IBLK)

            @pl.when(k >= 1)
            def _():
                wait_scatter(1)

            @pl.when(k < nblk - 1)
            def _():
                pltpu.sync_copy(eslice(src_hbm, (k + 1) * IBLK, IBLK),
                                src_v.at[pl.ds(no, IBLK)])
                pltpu.sync_copy(eslice(dst_hbm, (k + 1) * IBLK, IBLK),
                                dst_v.at[pl.ds(no, IBLK)])

            for j2 in range(IBLK):
                p = j2 & 1
                pn = p ^ 1
                if j2 >= 1:
                    wait_scatter(pn)
                if j2 < IBLK - 1:
                    fire_gather(src_v.at[co + j2 + 1], pn)
                else:
                    @pl.when(k < nblk - 1)
                    def _():
                        fire_gather(src_v.at[no], pn)
                wait_gather(p)
                fire_scatter(dst_v.at[co + j2], p)
            return carry

        lax.fori_loop(0, nblk, blk, 0)
        wait_scatter((nchunk - 1) & 1)
        plsc.subcore_barrier()
        pltpu.sync_copy(accum.at[pl.ds(s * RPT, RPT)],
                        out_hbm.at[c, pl.ds(s * RPT, RPT)])

        @pl.when(s == NS - 1)
        def _():
            pltpu.sync_copy(accum.at[pl.ds(NS * RPT, RTAIL)],
                            out_hbm.at[c, pl.ds(NS * RPT, RTAIL)])

    return agg_kernel


def _mlp_core(z_ref, w1_ref, mode, x_ref=None):
    if mode == "sumx":
        z = z_ref[0] + z_ref[1] - x_ref[...]
        return jnp.dot(z, w1_ref[...], preferred_element_type=jnp.float32)
    a = jnp.dot(z_ref[0], w1_ref[0:HALF, :],
                preferred_element_type=jnp.float32)
    a += jnp.dot(z_ref[1], w1_ref[HALF:2 * HALF, :],
                 preferred_element_type=jnp.float32)
    return a


def _make_mlp(d_in, mode):
    def body(*refs):
        if mode == "sumx":
            z_ref, x_ref, w1_ref, b1_ref, w2_ref, b2_ref, out_ref = refs
        else:
            z_ref, w1_ref, b1_ref, w2_ref, b2_ref, out_ref = refs
            x_ref = None
        a = jnp.maximum(_mlp_core(z_ref, w1_ref, mode, x_ref) + b1_ref[...],
                        0.0)
        h = jnp.dot(a, w2_ref[...], preferred_element_type=jnp.float32)
        h = jnp.maximum(h + b2_ref[...], 0.0)
        out_ref[0] = h[:, 0:HALF]
        out_ref[1] = h[:, HALF:2 * HALF]

    in_specs = [pl.BlockSpec((2, BLOCK_ROWS, HALF), lambda i: (0, i, 0))]
    if mode == "sumx":
        in_specs.append(pl.BlockSpec((BLOCK_ROWS, d_in), lambda i: (i, 0)))
    in_specs += [
        pl.BlockSpec((d_in, HIDDEN), lambda i: (0, 0)),
        pl.BlockSpec((1, HIDDEN), lambda i: (0, 0)),
        pl.BlockSpec((HIDDEN, HIDDEN), lambda i: (0, 0)),
        pl.BlockSpec((1, HIDDEN), lambda i: (0, 0)),
    ]
    return pl.pallas_call(
        body,
        grid=(GRID,),
        in_specs=in_specs,
        out_specs=pl.BlockSpec((2, BLOCK_ROWS, HALF), lambda i: (0, i, 0)),
        out_shape=jax.ShapeDtypeStruct((2, N_NODES, HALF), jnp.float32),
    )


def _mlp_pool_body(z_ref, w1_ref, b1_ref, w2_ref, b2_ref, batch_ref, g_ref,
                   sums_ref, cnts_ref):
    i = pl.program_id(0)

    @pl.when(i == 0)
    def _():
        sums_ref[...] = jnp.zeros_like(sums_ref)
        cnts_ref[...] = jnp.zeros_like(cnts_ref)

    a = jnp.maximum(_mlp_core(z_ref, w1_ref, "split") + b1_ref[...], 0.0)
    h = jnp.dot(a, w2_ref[...], preferred_element_type=jnp.float32)
    h = jnp.maximum(h + b2_ref[...], 0.0)

    b = batch_ref[0, 0, :]
    gids = lax.broadcasted_iota(jnp.int32, (N_GRAPHS, BLOCK_ROWS), 0)
    onehot = (b[None, :] == gids).astype(jnp.float32)
    sums_ref[...] += jnp.dot(onehot, h, preferred_element_type=jnp.float32)
    cnts_ref[...] += jnp.sum(onehot, axis=1, keepdims=True)

    @pl.when(i == GRID - 1)
    def _():
        g_ref[...] = sums_ref[...] / jnp.maximum(cnts_ref[:, :1], 1.0)


def _make_mlp_pool():
    return pl.pallas_call(
        _mlp_pool_body,
        grid=(GRID,),
        in_specs=[
            pl.BlockSpec((2, BLOCK_ROWS, HALF), lambda i: (0, i, 0)),
            pl.BlockSpec((HIDDEN, HIDDEN), lambda i: (0, 0)),
            pl.BlockSpec((1, HIDDEN), lambda i: (0, 0)),
            pl.BlockSpec((HIDDEN, HIDDEN), lambda i: (0, 0)),
            pl.BlockSpec((1, HIDDEN), lambda i: (0, 0)),
            pl.BlockSpec((1, 1, BLOCK_ROWS), lambda i: (i, 0, 0)),
        ],
        out_specs=pl.BlockSpec((N_GRAPHS, HIDDEN), lambda i: (0, 0)),
        out_shape=jax.ShapeDtypeStruct((N_GRAPHS, HIDDEN), jnp.float32),
        scratch_shapes=[
            pltpu.VMEM((N_GRAPHS, HIDDEN), jnp.float32),
            pltpu.VMEM((N_GRAPHS, 1), jnp.float32),
        ],
    )


_AGG0 = _make_agg(NCHUNK0, split_edges=True)
_AGG12 = _make_agg(NCHUNK12, split_edges=False)
_MLP0 = _make_mlp(128, "sumx")
_MLP1 = _make_mlp(256, "split")
_MLP_POOL = _make_mlp_pool()


def kernel(x, edge_index, batch,
           l0_w1, l0_b1, l0_w2, l0_b2,
           l1_w1, l1_b1, l1_w2, l1_b2,
           l2_w1, l2_b1, l2_w2, l2_b2):
    ei = edge_index.astype(jnp.int32)
    src0 = ei[0].reshape(2, NS, NCHUNK0, CHUNK)
    dst0 = ei[1].reshape(2, NS, NCHUNK0, CHUNK)
    src12 = ei[0].reshape(NS, NCHUNK12, CHUNK)
    dst12 = ei[1].reshape(NS, NCHUNK12, CHUNK)
    batch_r = batch.astype(jnp.int32).reshape(GRID, 1, BLOCK_ROWS)

    p = _AGG0(x, x, src0, dst0)
    h = _MLP0(p, x, l0_w1, l0_b1.reshape(1, HIDDEN),
              l0_w2, l0_b2.reshape(1, HIDDEN))
    z = _AGG12(h, h, src12, dst12)
    h = _MLP1(z, l1_w1, l1_b1.reshape(1, HIDDEN),
              l1_w2, l1_b2.reshape(1, HIDDEN))
    z = _AGG12(h, h, src12, dst12)
    g = _MLP_POOL(z, l2_w1, l2_b1.reshape(1, HIDDEN),
                  l2_w2, l2_b2.reshape(1, HIDDEN), batch_r)
    return g


# speedup vs baseline: 9.7481x; 1.0269x over previous
import functools

import jax
import jax.numpy as jnp
from jax import lax
from jax.experimental import pallas as pl
from jax.experimental.pallas import tpu as pltpu
from jax.experimental.pallas import tpu_sc as plsc

N_NODES = 10000
N_EDGES = 320000
N_GRAPHS = 64
HIDDEN = 256
HALF = 128

NS = 16
CHUNK = 125
RPT = 624
RTAIL = N_NODES - NS * RPT

NCHUNK0 = N_EDGES // 2 // NS // CHUNK
NCHUNK12 = N_EDGES // NS // CHUNK
IBLK = 16

BLOCK_ROWS = 1000
GRID = N_NODES // BLOCK_ROWS


def _make_agg(nchunk, split_edges, ib):
    mesh = plsc.VectorSubcoreMesh(core_axis_name="c", subcore_axis_name="s")

    nblk = nchunk // ib

    @functools.partial(
        pl.kernel,
        mesh=mesh,
        out_type=jax.ShapeDtypeStruct((2, N_NODES, HALF), jnp.float32),
        scratch_types=[
            pltpu.VMEM((2 * ib, CHUNK), jnp.int32),
            pltpu.VMEM((2 * ib, CHUNK), jnp.int32),
            pltpu.VMEM((2, CHUNK, HALF), jnp.float32),
            pltpu.VMEM_SHARED((N_NODES, HALF), jnp.float32),
            pltpu.SemaphoreType.DMA,
            pltpu.SemaphoreType.DMA,
            pltpu.SemaphoreType.DMA,
            pltpu.SemaphoreType.DMA,
        ],
    )
    def agg_kernel(table_hbm, init_hbm, src_hbm, dst_hbm, out_hbm,
                   src_v, dst_v, buf, accum, gsem0, gsem1, ssem0, ssem1):
        c = lax.axis_index("c")
        s = lax.axis_index("s")
        gsem = (gsem0, gsem1)
        ssem = (ssem0, ssem1)
        tbl = table_hbm if split_edges else table_hbm.at[c]
        ini = init_hbm if split_edges else init_hbm.at[c]

        def eslice(ref, lo, n):
            if split_edges:
                return ref.at[c, s, pl.ds(lo, n)]
            return ref.at[s, pl.ds(lo, n)]

        def fire_gather(idx_row, p):
            pltpu.async_copy(tbl.at[idx_row], buf.at[p], gsem[p])

        def wait_gather(p):
            pltpu.make_async_copy(tbl.at[src_v.at[0]], buf.at[p],
                                  gsem[p]).wait()

        def fire_scatter(idx_row, p):
            pltpu.async_copy(buf.at[p], accum.at[idx_row], ssem[p],
                             add=True)

        def wait_scatter(p):
            pltpu.make_async_copy(buf.at[p], accum.at[dst_v.at[0]],
                                  ssem[p]).wait()

        pltpu.sync_copy(eslice(src_hbm, 0, ib), src_v.at[pl.ds(0, ib)])
        pltpu.sync_copy(eslice(dst_hbm, 0, ib), dst_v.at[pl.ds(0, ib)])
        fire_gather(src_v.at[0], 0)
        pltpu.sync_copy(ini.at[pl.ds(s * RPT, RPT)],
                        accum.at[pl.ds(s * RPT, RPT)])

        @pl.when(s == NS - 1)
        def _():
            pltpu.sync_copy(ini.at[pl.ds(NS * RPT, RTAIL)],
                            accum.at[pl.ds(NS * RPT, RTAIL)])

        plsc.subcore_barrier()

        def blk(k, carry):
            co = pl.multiple_of((k % 2) * ib, ib)
            no = pl.multiple_of(ib - co, ib)

            @pl.when(k >= 1)
            def _():
                wait_scatter(1)

            @pl.when(k < nblk - 1)
            def _():
                pltpu.sync_copy(eslice(src_hbm, (k + 1) * ib, ib),
                                src_v.at[pl.ds(no, ib)])
                pltpu.sync_copy(eslice(dst_hbm, (k + 1) * ib, ib),
                                dst_v.at[pl.ds(no, ib)])

            for j2 in range(ib):
                p = j2 & 1
                pn = p ^ 1
                if j2 >= 1:
                    wait_scatter(pn)
                if j2 < ib - 1:
                    fire_gather(src_v.at[co + j2 + 1], pn)
                else:
                    @pl.when(k < nblk - 1)
                    def _():
                        fire_gather(src_v.at[no], pn)
                wait_gather(p)
                fire_scatter(dst_v.at[co + j2], p)
            return carry

        lax.fori_loop(0, nblk, blk, 0)
        wait_scatter((nchunk - 1) & 1)
        plsc.subcore_barrier()
        pltpu.sync_copy(accum.at[pl.ds(s * RPT, RPT)],
                        out_hbm.at[c, pl.ds(s * RPT, RPT)])

        @pl.when(s == NS - 1)
        def _():
            pltpu.sync_copy(accum.at[pl.ds(NS * RPT, RTAIL)],
                            out_hbm.at[c, pl.ds(NS * RPT, RTAIL)])

    return agg_kernel


def _mlp_core(z_ref, w1_ref, mode, x_ref=None):
    if mode == "sumx":
        z = z_ref[0] + z_ref[1] - x_ref[...]
        return jnp.dot(z, w1_ref[...], preferred_element_type=jnp.float32)
    a = jnp.dot(z_ref[0], w1_ref[0:HALF, :],
                preferred_element_type=jnp.float32)
    a += jnp.dot(z_ref[1], w1_ref[HALF:2 * HALF, :],
                 preferred_element_type=jnp.float32)
    return a


def _make_mlp(d_in, mode):
    def body(*refs):
        if mode == "sumx":
            z_ref, x_ref, w1_ref, b1_ref, w2_ref, b2_ref, out_ref = refs
        else:
            z_ref, w1_ref, b1_ref, w2_ref, b2_ref, out_ref = refs
            x_ref = None
        a = jnp.maximum(_mlp_core(z_ref, w1_ref, mode, x_ref) + b1_ref[...],
                        0.0)
        h = jnp.dot(a, w2_ref[...], preferred_element_type=jnp.float32)
        h = jnp.maximum(h + b2_ref[...], 0.0)
        out_ref[0] = h[:, 0:HALF]
        out_ref[1] = h[:, HALF:2 * HALF]

    in_specs = [pl.BlockSpec((2, BLOCK_ROWS, HALF), lambda i: (0, i, 0))]
    if mode == "sumx":
        in_specs.append(pl.BlockSpec((BLOCK_ROWS, d_in), lambda i: (i, 0)))
    in_specs += [
        pl.BlockSpec((d_in, HIDDEN), lambda i: (0, 0)),
        pl.BlockSpec((1, HIDDEN), lambda i: (0, 0)),
        pl.BlockSpec((HIDDEN, HIDDEN), lambda i: (0, 0)),
        pl.BlockSpec((1, HIDDEN), lambda i: (0, 0)),
    ]
    return pl.pallas_call(
        body,
        grid=(GRID,),
        in_specs=in_specs,
        out_specs=pl.BlockSpec((2, BLOCK_ROWS, HALF), lambda i: (0, i, 0)),
        out_shape=jax.ShapeDtypeStruct((2, N_NODES, HALF), jnp.float32),
    )


def _mlp_pool_body(z_ref, w1_ref, b1_ref, w2_ref, b2_ref, batch_ref, g_ref,
                   sums_ref, cnts_ref):
    i = pl.program_id(0)

    @pl.when(i == 0)
    def _():
        sums_ref[...] = jnp.zeros_like(sums_ref)
        cnts_ref[...] = jnp.zeros_like(cnts_ref)

    a = jnp.maximum(_mlp_core(z_ref, w1_ref, "split") + b1_ref[...], 0.0)
    h = jnp.dot(a, w2_ref[...], preferred_element_type=jnp.float32)
    h = jnp.maximum(h + b2_ref[...], 0.0)

    b = batch_ref[0, 0, :]
    gids = lax.broadcasted_iota(jnp.int32, (N_GRAPHS, BLOCK_ROWS), 0)
    onehot = (b[None, :] == gids).astype(jnp.float32)
    sums_ref[...] += jnp.dot(onehot, h, preferred_element_type=jnp.float32)
    cnts_ref[...] += jnp.sum(onehot, axis=1, keepdims=True)

    @pl.when(i == GRID - 1)
    def _():
        g_ref[...] = sums_ref[...] / jnp.maximum(cnts_ref[:, :1], 1.0)


def _make_mlp_pool():
    return pl.pallas_call(
        _mlp_pool_body,
        grid=(GRID,),
        in_specs=[
            pl.BlockSpec((2, BLOCK_ROWS, HALF), lambda i: (0, i, 0)),
            pl.BlockSpec((HIDDEN, HIDDEN), lambda i: (0, 0)),
            pl.BlockSpec((1, HIDDEN), lambda i: (0, 0)),
            pl.BlockSpec((HIDDEN, HIDDEN), lambda i: (0, 0)),
            pl.BlockSpec((1, HIDDEN), lambda i: (0, 0)),
            pl.BlockSpec((1, 1, BLOCK_ROWS), lambda i: (i, 0, 0)),
        ],
        out_specs=pl.BlockSpec((N_GRAPHS, HIDDEN), lambda i: (0, 0)),
        out_shape=jax.ShapeDtypeStruct((N_GRAPHS, HIDDEN), jnp.float32),
        scratch_shapes=[
            pltpu.VMEM((N_GRAPHS, HIDDEN), jnp.float32),
            pltpu.VMEM((N_GRAPHS, 1), jnp.float32),
        ],
    )


_AGG0 = _make_agg(NCHUNK0, split_edges=True, ib=16)
_AGG12 = _make_agg(NCHUNK12, split_edges=False, ib=32)
_MLP0 = _make_mlp(128, "sumx")
_MLP1 = _make_mlp(256, "split")
_MLP_POOL = _make_mlp_pool()


def kernel(x, edge_index, batch,
           l0_w1, l0_b1, l0_w2, l0_b2,
           l1_w1, l1_b1, l1_w2, l1_b2,
           l2_w1, l2_b1, l2_w2, l2_b2):
    ei = edge_index.astype(jnp.int32)
    src0 = ei[0].reshape(2, NS, NCHUNK0, CHUNK)
    dst0 = ei[1].reshape(2, NS, NCHUNK0, CHUNK)
    src12 = ei[0].reshape(NS, NCHUNK12, CHUNK)
    dst12 = ei[1].reshape(NS, NCHUNK12, CHUNK)
    batch_r = batch.astype(jnp.int32).reshape(GRID, 1, BLOCK_ROWS)

    p = _AGG0(x, x, src0, dst0)
    h = _MLP0(p, x, l0_w1, l0_b1.reshape(1, HIDDEN),
              l0_w2, l0_b2.reshape(1, HIDDEN))
    z = _AGG12(h, h, src12, dst12)
    h = _MLP1(z, l1_w1, l1_b1.reshape(1, HIDDEN),
              l1_w2, l1_b2.reshape(1, HIDDEN))
    z = _AGG12(h, h, src12, dst12)
    g = _MLP_POOL(z, l2_w1, l2_b1.reshape(1, HIDDEN),
                  l2_w2, l2_b2.reshape(1, HIDDEN), batch_r)
    return g


# speedup vs baseline: 10.0971x; 1.0358x over previous
import functools

import jax
import jax.numpy as jnp
from jax import lax
from jax.experimental import pallas as pl
from jax.experimental.pallas import tpu as pltpu
from jax.experimental.pallas import tpu_sc as plsc

N_NODES = 10000
N_EDGES = 320000
N_GRAPHS = 64
HIDDEN = 256
HALF = 128

NS = 16
CHUNK = 125
RPT = 624
RTAIL = N_NODES - NS * RPT

NCHUNK0 = N_EDGES // 2 // NS // CHUNK
NCHUNK12 = N_EDGES // NS // CHUNK
IBLK = 16

BLOCK_ROWS = 1000
GRID = N_NODES // BLOCK_ROWS


def _make_agg(nchunk, split_edges, ib):
    mesh = plsc.VectorSubcoreMesh(core_axis_name="c", subcore_axis_name="s")

    nblk = nchunk // ib

    @functools.partial(
        pl.kernel,
        mesh=mesh,
        out_type=jax.ShapeDtypeStruct((2, N_NODES, HALF), jnp.float32),
        scratch_types=[
            pltpu.VMEM((2, ib, 2, CHUNK), jnp.int32),
            pltpu.VMEM((2, CHUNK, HALF), jnp.float32),
            pltpu.VMEM_SHARED((N_NODES, HALF), jnp.float32),
            pltpu.SemaphoreType.DMA,
            pltpu.SemaphoreType.DMA,
            pltpu.SemaphoreType.DMA,
            pltpu.SemaphoreType.DMA,
            pltpu.SemaphoreType.DMA,
        ],
    )
    def agg_kernel(table_hbm, init_hbm, eidx_hbm, out_hbm,
                   iv, buf, accum, gsem0, gsem1, ssem0, ssem1, isem):
        c = lax.axis_index("c")
        s = lax.axis_index("s")
        gsem = (gsem0, gsem1)
        ssem = (ssem0, ssem1)
        tbl = table_hbm if split_edges else table_hbm.at[c]
        ini = init_hbm if split_edges else init_hbm.at[c]

        def eslice(lo, n):
            if split_edges:
                return eidx_hbm.at[c, s, pl.ds(lo, n)]
            return eidx_hbm.at[s, pl.ds(lo, n)]

        def fire_gather(idx_row, p):
            pltpu.async_copy(tbl.at[idx_row], buf.at[p], gsem[p])

        def wait_gather(p):
            pltpu.make_async_copy(tbl.at[iv.at[0, 0, 0]], buf.at[p],
                                  gsem[p]).wait()

        def fire_scatter(idx_row, p):
            pltpu.async_copy(buf.at[p], accum.at[idx_row], ssem[p],
                             add=True)

        def wait_scatter(p):
            pltpu.make_async_copy(buf.at[p], accum.at[iv.at[0, 0, 1]],
                                  ssem[p]).wait()

        pltpu.sync_copy(eslice(0, ib), iv.at[0])
        fire_gather(iv.at[0, 0, 0], 0)
        pltpu.sync_copy(ini.at[pl.ds(s * RPT, RPT)],
                        accum.at[pl.ds(s * RPT, RPT)])

        @pl.when(s == NS - 1)
        def _():
            pltpu.sync_copy(ini.at[pl.ds(NS * RPT, RTAIL)],
                            accum.at[pl.ds(NS * RPT, RTAIL)])

        plsc.subcore_barrier()

        def blk(k, carry):
            r = k % 2
            rn = 1 - r

            @pl.when(k >= 1)
            def _():
                wait_scatter(1)

            @pl.when(k < nblk - 1)
            def _():
                pltpu.async_copy(eslice((k + 1) * ib, ib), iv.at[rn], isem)

            for j2 in range(ib):
                p = j2 & 1
                pn = p ^ 1
                if j2 >= 1:
                    wait_scatter(pn)
                if j2 < ib - 1:
                    fire_gather(iv.at[r, j2 + 1, 0], pn)
                else:
                    @pl.when(k < nblk - 1)
                    def _():
                        pltpu.make_async_copy(eslice(0, ib), iv.at[rn],
                                              isem).wait()
                        fire_gather(iv.at[rn, 0, 0], pn)
                wait_gather(p)
                fire_scatter(iv.at[r, j2, 1], p)
            return carry

        lax.fori_loop(0, nblk, blk, 0)
        wait_scatter((nchunk - 1) & 1)
        plsc.subcore_barrier()
        pltpu.sync_copy(accum.at[pl.ds(s * RPT, RPT)],
                        out_hbm.at[c, pl.ds(s * RPT, RPT)])

        @pl.when(s == NS - 1)
        def _():
            pltpu.sync_copy(accum.at[pl.ds(NS * RPT, RTAIL)],
                            out_hbm.at[c, pl.ds(NS * RPT, RTAIL)])

    return agg_kernel


def _mlp_core(z_ref, w1_ref, mode, x_ref=None):
    if mode == "sumx":
        z = z_ref[0] + z_ref[1] - x_ref[...]
        return jnp.dot(z, w1_ref[...], preferred_element_type=jnp.float32)
    a = jnp.dot(z_ref[0], w1_ref[0:HALF, :],
                preferred_element_type=jnp.float32)
    a += jnp.dot(z_ref[1], w1_ref[HALF:2 * HALF, :],
                 preferred_element_type=jnp.float32)
    return a


def _make_mlp(d_in, mode):
    def body(*refs):
        if mode == "sumx":
            z_ref, x_ref, w1_ref, b1_ref, w2_ref, b2_ref, out_ref = refs
        else:
            z_ref, w1_ref, b1_ref, w2_ref, b2_ref, out_ref = refs
            x_ref = None
        a = jnp.maximum(_mlp_core(z_ref, w1_ref, mode, x_ref) + b1_ref[...],
                        0.0)
        h = jnp.dot(a, w2_ref[...], preferred_element_type=jnp.float32)
        h = jnp.maximum(h + b2_ref[...], 0.0)
        out_ref[0] = h[:, 0:HALF]
        out_ref[1] = h[:, HALF:2 * HALF]

    in_specs = [pl.BlockSpec((2, BLOCK_ROWS, HALF), lambda i: (0, i, 0))]
    if mode == "sumx":
        in_specs.append(pl.BlockSpec((BLOCK_ROWS, d_in), lambda i: (i, 0)))
    in_specs += [
        pl.BlockSpec((d_in, HIDDEN), lambda i: (0, 0)),
        pl.BlockSpec((1, HIDDEN), lambda i: (0, 0)),
        pl.BlockSpec((HIDDEN, HIDDEN), lambda i: (0, 0)),
        pl.BlockSpec((1, HIDDEN), lambda i: (0, 0)),
    ]
    return pl.pallas_call(
        body,
        grid=(GRID,),
        in_specs=in_specs,
        out_specs=pl.BlockSpec((2, BLOCK_ROWS, HALF), lambda i: (0, i, 0)),
        out_shape=jax.ShapeDtypeStruct((2, N_NODES, HALF), jnp.float32),
    )


def _mlp_pool_body(z_ref, w1_ref, b1_ref, w2_ref, b2_ref, batch_ref, g_ref,
                   sums_ref, cnts_ref):
    i = pl.program_id(0)

    @pl.when(i == 0)
    def _():
        sums_ref[...] = jnp.zeros_like(sums_ref)
        cnts_ref[...] = jnp.zeros_like(cnts_ref)

    a = jnp.maximum(_mlp_core(z_ref, w1_ref, "split") + b1_ref[...], 0.0)
    h = jnp.dot(a, w2_ref[...], preferred_element_type=jnp.float32)
    h = jnp.maximum(h + b2_ref[...], 0.0)

    b = batch_ref[0, 0, :]
    gids = lax.broadcasted_iota(jnp.int32, (N_GRAPHS, BLOCK_ROWS), 0)
    onehot = (b[None, :] == gids).astype(jnp.float32)
    sums_ref[...] += jnp.dot(onehot, h, preferred_element_type=jnp.float32)
    cnts_ref[...] += jnp.sum(onehot, axis=1, keepdims=True)

    @pl.when(i == GRID - 1)
    def _():
        g_ref[...] = sums_ref[...] / jnp.maximum(cnts_ref[:, :1], 1.0)


def _make_mlp_pool():
    return pl.pallas_call(
        _mlp_pool_body,
        grid=(GRID,),
        in_specs=[
            pl.BlockSpec((2, BLOCK_ROWS, HALF), lambda i: (0, i, 0)),
            pl.BlockSpec((HIDDEN, HIDDEN), lambda i: (0, 0)),
            pl.BlockSpec((1, HIDDEN), lambda i: (0, 0)),
            pl.BlockSpec((HIDDEN, HIDDEN), lambda i: (0, 0)),
            pl.BlockSpec((1, HIDDEN), lambda i: (0, 0)),
            pl.BlockSpec((1, 1, BLOCK_ROWS), lambda i: (i, 0, 0)),
        ],
        out_specs=pl.BlockSpec((N_GRAPHS, HIDDEN), lambda i: (0, 0)),
        out_shape=jax.ShapeDtypeStruct((N_GRAPHS, HIDDEN), jnp.float32),
        scratch_shapes=[
            pltpu.VMEM((N_GRAPHS, HIDDEN), jnp.float32),
            pltpu.VMEM((N_GRAPHS, 1), jnp.float32),
        ],
    )


_AGG0 = _make_agg(NCHUNK0, split_edges=True, ib=16)
_AGG12 = _make_agg(NCHUNK12, split_edges=False, ib=32)
_MLP0 = _make_mlp(128, "sumx")
_MLP1 = _make_mlp(256, "split")
_MLP_POOL = _make_mlp_pool()


def kernel(x, edge_index, batch,
           l0_w1, l0_b1, l0_w2, l0_b2,
           l1_w1, l1_b1, l1_w2, l1_b2,
           l2_w1, l2_b1, l2_w2, l2_b2):
    ei = edge_index.astype(jnp.int32)
    e0 = ei.reshape(2, 2, NS, NCHUNK0, CHUNK).transpose(1, 2, 3, 0, 4)
    e12 = ei.reshape(2, NS, NCHUNK12, CHUNK).transpose(1, 2, 0, 3)
    batch_r = batch.astype(jnp.int32).reshape(GRID, 1, BLOCK_ROWS)

    p = _AGG0(x, x, e0)
    h = _MLP0(p, x, l0_w1, l0_b1.reshape(1, HIDDEN),
              l0_w2, l0_b2.reshape(1, HIDDEN))
    z = _AGG12(h, h, e12)
    h = _MLP1(z, l1_w1, l1_b1.reshape(1, HIDDEN),
              l1_w2, l1_b2.reshape(1, HIDDEN))
    z = _AGG12(h, h, e12)
    g = _MLP_POOL(z, l2_w1, l2_b1.reshape(1, HIDDEN),
                  l2_w2, l2_b2.reshape(1, HIDDEN), batch_r)
    return g


# speedup vs baseline: 10.2325x; 1.0134x over previous
import functools

import jax
import jax.numpy as jnp
from jax import lax
from jax.experimental import pallas as pl
from jax.experimental.pallas import tpu as pltpu
from jax.experimental.pallas import tpu_sc as plsc

N_NODES = 10000
N_EDGES = 320000
N_GRAPHS = 64
HIDDEN = 256
HALF = 128

NS = 16
CHUNK = 125
RPT = 624
RTAIL = N_NODES - NS * RPT

NCHUNK0 = N_EDGES // 2 // NS // CHUNK
NCHUNK12 = N_EDGES // NS // CHUNK
IBLK = 16

BLOCK_ROWS = 2000
GRID = N_NODES // BLOCK_ROWS


def _make_agg(nchunk, split_edges, ib):
    mesh = plsc.VectorSubcoreMesh(core_axis_name="c", subcore_axis_name="s")

    nblk = nchunk // ib

    @functools.partial(
        pl.kernel,
        mesh=mesh,
        out_type=jax.ShapeDtypeStruct((2, N_NODES, HALF), jnp.float32),
        scratch_types=[
            pltpu.VMEM((2, ib, 2, CHUNK), jnp.int32),
            pltpu.VMEM((2, CHUNK, HALF), jnp.float32),
            pltpu.VMEM_SHARED((N_NODES, HALF), jnp.float32),
            pltpu.SemaphoreType.DMA,
            pltpu.SemaphoreType.DMA,
            pltpu.SemaphoreType.DMA,
            pltpu.SemaphoreType.DMA,
            pltpu.SemaphoreType.DMA,
        ],
    )
    def agg_kernel(table_hbm, init_hbm, eidx_hbm, out_hbm,
                   iv, buf, accum, gsem0, gsem1, ssem0, ssem1, isem):
        c = lax.axis_index("c")
        s = lax.axis_index("s")
        gsem = (gsem0, gsem1)
        ssem = (ssem0, ssem1)
        tbl = table_hbm if split_edges else table_hbm.at[c]
        ini = init_hbm if split_edges else init_hbm.at[c]

        def eslice(lo, n):
            if split_edges:
                return eidx_hbm.at[c, s, pl.ds(lo, n)]
            return eidx_hbm.at[s, pl.ds(lo, n)]

        def fire_gather(idx_row, p):
            pltpu.async_copy(tbl.at[idx_row], buf.at[p], gsem[p])

        def wait_gather(p):
            pltpu.make_async_copy(tbl.at[iv.at[0, 0, 0]], buf.at[p],
                                  gsem[p]).wait()

        def fire_scatter(idx_row, p):
            pltpu.async_copy(buf.at[p], accum.at[idx_row], ssem[p],
                             add=True)

        def wait_scatter(p):
            pltpu.make_async_copy(buf.at[p], accum.at[iv.at[0, 0, 1]],
                                  ssem[p]).wait()

        pltpu.sync_copy(eslice(0, ib), iv.at[0])
        fire_gather(iv.at[0, 0, 0], 0)
        pltpu.sync_copy(ini.at[pl.ds(s * RPT, RPT)],
                        accum.at[pl.ds(s * RPT, RPT)])

        @pl.when(s == NS - 1)
        def _():
            pltpu.sync_copy(ini.at[pl.ds(NS * RPT, RTAIL)],
                            accum.at[pl.ds(NS * RPT, RTAIL)])

        plsc.subcore_barrier()

        def blk(k, carry):
            r = k % 2
            rn = 1 - r

            @pl.when(k >= 1)
            def _():
                wait_scatter(1)

            @pl.when(k < nblk - 1)
            def _():
                pltpu.async_copy(eslice((k + 1) * ib, ib), iv.at[rn], isem)

            for j2 in range(ib):
                p = j2 & 1
                pn = p ^ 1
                if j2 >= 1:
                    wait_scatter(pn)
                if j2 < ib - 1:
                    fire_gather(iv.at[r, j2 + 1, 0], pn)
                else:
                    @pl.when(k < nblk - 1)
                    def _():
                        pltpu.make_async_copy(eslice(0, ib), iv.at[rn],
                                              isem).wait()
                        fire_gather(iv.at[rn, 0, 0], pn)
                wait_gather(p)
                fire_scatter(iv.at[r, j2, 1], p)
            return carry

        lax.fori_loop(0, nblk, blk, 0)
        wait_scatter((nchunk - 1) & 1)
        plsc.subcore_barrier()
        pltpu.sync_copy(accum.at[pl.ds(s * RPT, RPT)],
                        out_hbm.at[c, pl.ds(s * RPT, RPT)])

        @pl.when(s == NS - 1)
        def _():
            pltpu.sync_copy(accum.at[pl.ds(NS * RPT, RTAIL)],
                            out_hbm.at[c, pl.ds(NS * RPT, RTAIL)])

    return agg_kernel


def _mlp_core(z_ref, w1_ref, mode, x_ref=None):
    if mode == "sumx":
        z = z_ref[0] + z_ref[1] - x_ref[...]
        return jnp.dot(z, w1_ref[...], preferred_element_type=jnp.float32)
    a = jnp.dot(z_ref[0], w1_ref[0:HALF, :],
                preferred_element_type=jnp.float32)
    a += jnp.dot(z_ref[1], w1_ref[HALF:2 * HALF, :],
                 preferred_element_type=jnp.float32)
    return a


def _make_mlp(d_in, mode):
    def body(*refs):
        if mode == "sumx":
            z_ref, x_ref, w1_ref, b1_ref, w2_ref, b2_ref, out_ref = refs
        else:
            z_ref, w1_ref, b1_ref, w2_ref, b2_ref, out_ref = refs
            x_ref = None
        a = jnp.maximum(_mlp_core(z_ref, w1_ref, mode, x_ref) + b1_ref[...],
                        0.0)
        h = jnp.dot(a, w2_ref[...], preferred_element_type=jnp.float32)
        h = jnp.maximum(h + b2_ref[...], 0.0)
        out_ref[0] = h[:, 0:HALF]
        out_ref[1] = h[:, HALF:2 * HALF]

    in_specs = [pl.BlockSpec((2, BLOCK_ROWS, HALF), lambda i: (0, i, 0))]
    if mode == "sumx":
        in_specs.append(pl.BlockSpec((BLOCK_ROWS, d_in), lambda i: (i, 0)))
    in_specs += [
        pl.BlockSpec((d_in, HIDDEN), lambda i: (0, 0)),
        pl.BlockSpec((1, HIDDEN), lambda i: (0, 0)),
        pl.BlockSpec((HIDDEN, HIDDEN), lambda i: (0, 0)),
        pl.BlockSpec((1, HIDDEN), lambda i: (0, 0)),
    ]
    return pl.pallas_call(
        body,
        grid=(GRID,),
        in_specs=in_specs,
        out_specs=pl.BlockSpec((2, BLOCK_ROWS, HALF), lambda i: (0, i, 0)),
        out_shape=jax.ShapeDtypeStruct((2, N_NODES, HALF), jnp.float32),
    )


def _mlp_pool_body(z_ref, w1_ref, b1_ref, w2_ref, b2_ref, batch_ref, g_ref,
                   sums_ref, cnts_ref):
    i = pl.program_id(0)

    @pl.when(i == 0)
    def _():
        sums_ref[...] = jnp.zeros_like(sums_ref)
        cnts_ref[...] = jnp.zeros_like(cnts_ref)

    a = jnp.maximum(_mlp_core(z_ref, w1_ref, "split") + b1_ref[...], 0.0)
    h = jnp.dot(a, w2_ref[...], preferred_element_type=jnp.float32)
    h = jnp.maximum(h + b2_ref[...], 0.0)

    b = batch_ref[0, 0, :]
    gids = lax.broadcasted_iota(jnp.int32, (N_GRAPHS, BLOCK_ROWS), 0)
    onehot = (b[None, :] == gids).astype(jnp.float32)
    sums_ref[...] += jnp.dot(onehot, h, preferred_element_type=jnp.float32)
    cnts_ref[...] += jnp.sum(onehot, axis=1, keepdims=True)

    @pl.when(i == GRID - 1)
    def _():
        g_ref[...] = sums_ref[...] / jnp.maximum(cnts_ref[:, :1], 1.0)


def _make_mlp_pool():
    return pl.pallas_call(
        _mlp_pool_body,
        grid=(GRID,),
        in_specs=[
            pl.BlockSpec((2, BLOCK_ROWS, HALF), lambda i: (0, i, 0)),
            pl.BlockSpec((HIDDEN, HIDDEN), lambda i: (0, 0)),
            pl.BlockSpec((1, HIDDEN), lambda i: (0, 0)),
            pl.BlockSpec((HIDDEN, HIDDEN), lambda i: (0, 0)),
            pl.BlockSpec((1, HIDDEN), lambda i: (0, 0)),
            pl.BlockSpec((1, 1, BLOCK_ROWS), lambda i: (i, 0, 0)),
        ],
        out_specs=pl.BlockSpec((N_GRAPHS, HIDDEN), lambda i: (0, 0)),
        out_shape=jax.ShapeDtypeStruct((N_GRAPHS, HIDDEN), jnp.float32),
        scratch_shapes=[
            pltpu.VMEM((N_GRAPHS, HIDDEN), jnp.float32),
            pltpu.VMEM((N_GRAPHS, 1), jnp.float32),
        ],
    )


_AGG0 = _make_agg(NCHUNK0, split_edges=True, ib=16)
_AGG12 = _make_agg(NCHUNK12, split_edges=False, ib=32)
_MLP0 = _make_mlp(128, "sumx")
_MLP1 = _make_mlp(256, "split")
_MLP_POOL = _make_mlp_pool()


def kernel(x, edge_index, batch,
           l0_w1, l0_b1, l0_w2, l0_b2,
           l1_w1, l1_b1, l1_w2, l1_b2,
           l2_w1, l2_b1, l2_w2, l2_b2):
    ei = edge_index.astype(jnp.int32)
    e0 = ei.reshape(2, 2, NS, NCHUNK0, CHUNK).transpose(1, 2, 3, 0, 4)
    e12 = ei.reshape(2, NS, NCHUNK12, CHUNK).transpose(1, 2, 0, 3)
    batch_r = batch.astype(jnp.int32).reshape(GRID, 1, BLOCK_ROWS)

    p = _AGG0(x, x, e0)
    h = _MLP0(p, x, l0_w1, l0_b1.reshape(1, HIDDEN),
              l0_w2, l0_b2.reshape(1, HIDDEN))
    z = _AGG12(h, h, e12)
    h = _MLP1(z, l1_w1, l1_b1.reshape(1, HIDDEN),
              l1_w2, l1_b2.reshape(1, HIDDEN))
    z = _AGG12(h, h, e12)
    g = _MLP_POOL(z, l2_w1, l2_b1.reshape(1, HIDDEN),
                  l2_w2, l2_b2.reshape(1, HIDDEN), batch_r)
    return g
